# Initial kernel scaffold; baseline (speedup 1.0000x reference)
import jax
import jax.numpy as jnp
from jax.experimental import pallas as pl


def kernel(bbox_targets, bbox_weights, labels):
    raise NotImplementedError("write your pallas kernel here")


# speedup vs baseline: 4.0997x; 4.0997x over previous
import jax
import jax.numpy as jnp
from jax.experimental import pallas as pl


def _copy_body(a_ref, b_ref, oa_ref, ob_ref):
    oa_ref[...] = a_ref[...]
    ob_ref[...] = b_ref[...]


def kernel(bbox_targets, bbox_weights, labels):
    M, N = bbox_targets.shape
    flat_t = bbox_targets.reshape(-1, 128)
    flat_w = bbox_weights.reshape(-1, 128)
    R = flat_t.shape[0]
    BR = 8192
    grid = pl.cdiv(R, BR)
    spec = pl.BlockSpec((BR, 128), lambda i: (i, 0))
    out_t, out_w = pl.pallas_call(
        _copy_body,
        grid=(grid,),
        in_specs=[spec, spec],
        out_specs=[spec, spec],
        out_shape=[jax.ShapeDtypeStruct((R, 128), jnp.float32)] * 2,
    )(flat_t, flat_w)
    return out_t.reshape(M, N), out_w.reshape(M, N)
